# Initial kernel scaffold; baseline (speedup 1.0000x reference)
import functools

import jax
import jax.numpy as jnp
from jax import lax
from jax.experimental import pallas as pl
from jax.experimental.pallas import tpu as pltpu
from jax.experimental.pallas import tpu_sc as plsc

NUM_EMBEDDINGS = 1000000
EMBED_DIM = 32
BATCH = 16384
N_FIELDS = 26

B_TOTAL = BATCH * N_FIELDS
NUM_CORES = 2
NUM_SUBCORES = 16
NW = NUM_CORES * NUM_SUBCORES
BPW = B_TOTAL // NW
CHUNK = 128
NCH = BPW // CHUNK


@functools.partial(
    pl.kernel,
    out_type=jax.ShapeDtypeStruct((B_TOTAL, EMBED_DIM), jnp.float32),
    mesh=plsc.VectorSubcoreMesh(core_axis_name="c", subcore_axis_name="s"),
    scratch_types=[
        pltpu.VMEM((NCH, CHUNK), jnp.int32),
        pltpu.VMEM((CHUNK, EMBED_DIM), jnp.float32),
        pltpu.SemaphoreType.DMA,
    ],
)
def _emb_gather(idx_hbm, table_hbm, out_hbm, idx_v, rows_v, sem):
    wid = lax.axis_index("s") * NUM_CORES + lax.axis_index("c")
    pltpu.sync_copy(idx_hbm.at[wid], idx_v)
    base = wid * BPW

    def body(j, _):
        pltpu.async_copy(table_hbm.at[idx_v.at[j]], rows_v, sem).wait()
        pltpu.sync_copy(rows_v, out_hbm.at[pl.ds(base + j * CHUNK, CHUNK)])
        return ()

    lax.fori_loop(0, NCH, body, ())


def kernel(indices, table):
    idx3 = indices.reshape(NW, NCH, CHUNK)
    out = _emb_gather(idx3, table)
    return out.reshape(BATCH, N_FIELDS, EMBED_DIM)


# speedup vs baseline: 1.4368x; 1.4368x over previous
import functools

import jax
import jax.numpy as jnp
from jax import lax
from jax.experimental import pallas as pl
from jax.experimental.pallas import tpu as pltpu
from jax.experimental.pallas import tpu_sc as plsc

NUM_EMBEDDINGS = 1000000
EMBED_DIM = 32
BATCH = 16384
N_FIELDS = 26

B_TOTAL = BATCH * N_FIELDS
NUM_CORES = 2
NUM_SUBCORES = 16
NW = NUM_CORES * NUM_SUBCORES
BPW = B_TOTAL // NW
CHUNK = 128
NCH = BPW // CHUNK


@functools.partial(
    pl.kernel,
    out_type=jax.ShapeDtypeStruct((B_TOTAL, EMBED_DIM), jnp.float32),
    mesh=plsc.VectorSubcoreMesh(core_axis_name="c", subcore_axis_name="s"),
    scratch_types=[
        pltpu.VMEM((NCH, CHUNK), jnp.int32),
        pltpu.VMEM((CHUNK, EMBED_DIM), jnp.float32),
        pltpu.SemaphoreType.DMA,
    ],
    compiler_params=pltpu.CompilerParams(use_tc_tiling_on_sc=False),
)
def _emb_gather(idx_hbm, table_hbm, out_hbm, idx_v, rows_v, sem):
    wid = lax.axis_index("s") * NUM_CORES + lax.axis_index("c")
    pltpu.sync_copy(idx_hbm.at[wid], idx_v)
    base = wid * BPW

    def body(j, _):
        pltpu.async_copy(table_hbm.at[idx_v.at[j]], rows_v, sem).wait()
        pltpu.sync_copy(rows_v, out_hbm.at[pl.ds(base + j * CHUNK, CHUNK)])
        return ()

    lax.fori_loop(0, NCH, body, ())


def kernel(indices, table):
    idx3 = indices.reshape(NW, NCH, CHUNK)
    out = _emb_gather(idx3, table)
    return out.reshape(BATCH, N_FIELDS, EMBED_DIM)


# speedup vs baseline: 1.5751x; 1.0962x over previous
import functools

import jax
import jax.numpy as jnp
from jax import lax
from jax.experimental import pallas as pl
from jax.experimental.pallas import tpu as pltpu
from jax.experimental.pallas import tpu_sc as plsc

NUM_EMBEDDINGS = 1000000
EMBED_DIM = 32
BATCH = 16384
N_FIELDS = 26

B_TOTAL = BATCH * N_FIELDS
NUM_CORES = 2
NUM_SUBCORES = 16
NW = NUM_CORES * NUM_SUBCORES
BPW = B_TOTAL // NW
CHUNK = 128
NCH = BPW // CHUNK
S = 13
SUPER = S * CHUNK
NSUP = NCH // S


@functools.partial(
    pl.kernel,
    out_type=jax.ShapeDtypeStruct((B_TOTAL, EMBED_DIM), jnp.float32),
    mesh=plsc.VectorSubcoreMesh(core_axis_name="c", subcore_axis_name="s"),
    scratch_types=[
        pltpu.VMEM((NCH, CHUNK), jnp.int32),
        pltpu.VMEM((2, SUPER, EMBED_DIM), jnp.float32),
        pltpu.SemaphoreType.DMA,
        pltpu.SemaphoreType.DMA,
        pltpu.SemaphoreType.DMA,
    ],
    compiler_params=pltpu.CompilerParams(use_tc_tiling_on_sc=False),
)
def _emb_gather(idx_hbm, table_hbm, out_hbm, idx_v, rows_v, sem_g0, sem_g1,
                sem_o):
    wid = lax.axis_index("s") * NUM_CORES + lax.axis_index("c")
    pltpu.sync_copy(idx_hbm.at[wid], idx_v)
    base = wid * BPW

    def issue_gathers(g, buf, sem):
        for c in range(S):
            pltpu.async_copy(
                table_hbm.at[idx_v.at[g * S + c]],
                rows_v.at[buf, pl.ds(c * CHUNK, CHUNK)],
                sem)

    def drain_gathers(buf, sem):
        for c in range(S):
            pltpu.make_async_copy(
                table_hbm.at[idx_v.at[0]],
                rows_v.at[buf, pl.ds(c * CHUNK, CHUNK)],
                sem).wait()

    def issue_out(g, buf):
        pltpu.async_copy(rows_v.at[buf],
                         out_hbm.at[pl.ds(base + g * SUPER, SUPER)], sem_o)

    def wait_out():
        pltpu.make_async_copy(rows_v.at[0], out_hbm.at[pl.ds(0, SUPER)],
                              sem_o).wait()

    issue_gathers(0, 0, sem_g0)

    def pair(t, _):
        g0 = 2 * t

        @pl.when(g0 > 0)
        def _():
            wait_out()

        issue_gathers(g0 + 1, 1, sem_g1)
        drain_gathers(0, sem_g0)
        issue_out(g0, 0)

        g1 = g0 + 1
        wait_out()

        @pl.when(g1 + 1 < NSUP)
        def _():
            issue_gathers(g1 + 1, 0, sem_g0)

        drain_gathers(1, sem_g1)
        issue_out(g1, 1)
        return ()

    lax.fori_loop(0, NSUP // 2, pair, ())
    wait_out()


def kernel(indices, table):
    idx3 = indices.reshape(NW, NCH, CHUNK)
    out = _emb_gather(idx3, table)
    return out.reshape(BATCH, N_FIELDS, EMBED_DIM)


# speedup vs baseline: 1.5801x; 1.0032x over previous
import functools

import jax
import jax.numpy as jnp
from jax import lax
from jax.experimental import pallas as pl
from jax.experimental.pallas import tpu as pltpu
from jax.experimental.pallas import tpu_sc as plsc

NUM_EMBEDDINGS = 1000000
EMBED_DIM = 32
BATCH = 16384
N_FIELDS = 26

B_TOTAL = BATCH * N_FIELDS
NUM_CORES = 2
NUM_SUBCORES = 16
NW = NUM_CORES * NUM_SUBCORES
BPW = B_TOTAL // NW
CHUNK = 1664
NCH = BPW // CHUNK
S = 1
SUPER = S * CHUNK
NSUP = NCH // S


@functools.partial(
    pl.kernel,
    out_type=jax.ShapeDtypeStruct((B_TOTAL, EMBED_DIM), jnp.float32),
    mesh=plsc.VectorSubcoreMesh(core_axis_name="c", subcore_axis_name="s"),
    scratch_types=[
        pltpu.VMEM((NCH, CHUNK), jnp.int32),
        pltpu.VMEM((2, SUPER, EMBED_DIM), jnp.float32),
        pltpu.SemaphoreType.DMA,
        pltpu.SemaphoreType.DMA,
        pltpu.SemaphoreType.DMA,
    ],
    compiler_params=pltpu.CompilerParams(use_tc_tiling_on_sc=False),
)
def _emb_gather(idx_hbm, table_hbm, out_hbm, idx_v, rows_v, sem_g0, sem_g1,
                sem_o):
    wid = lax.axis_index("s") * NUM_CORES + lax.axis_index("c")
    pltpu.sync_copy(idx_hbm.at[wid], idx_v)
    base = wid * BPW

    def issue_gathers(g, buf, sem):
        for c in range(S):
            pltpu.async_copy(
                table_hbm.at[idx_v.at[g * S + c]],
                rows_v.at[buf, pl.ds(c * CHUNK, CHUNK)],
                sem)

    def drain_gathers(buf, sem):
        for c in range(S):
            pltpu.make_async_copy(
                table_hbm.at[idx_v.at[0]],
                rows_v.at[buf, pl.ds(c * CHUNK, CHUNK)],
                sem).wait()

    def issue_out(g, buf):
        pltpu.async_copy(rows_v.at[buf],
                         out_hbm.at[pl.ds(base + g * SUPER, SUPER)], sem_o)

    def wait_out():
        pltpu.make_async_copy(rows_v.at[0], out_hbm.at[pl.ds(0, SUPER)],
                              sem_o).wait()

    issue_gathers(0, 0, sem_g0)

    def pair(t, _):
        g0 = 2 * t

        @pl.when(g0 > 0)
        def _():
            wait_out()

        issue_gathers(g0 + 1, 1, sem_g1)
        drain_gathers(0, sem_g0)
        issue_out(g0, 0)

        g1 = g0 + 1
        wait_out()

        @pl.when(g1 + 1 < NSUP)
        def _():
            issue_gathers(g1 + 1, 0, sem_g0)

        drain_gathers(1, sem_g1)
        issue_out(g1, 1)
        return ()

    lax.fori_loop(0, NSUP // 2, pair, ())
    wait_out()


def kernel(indices, table):
    idx3 = indices.reshape(NW, NCH, CHUNK)
    out = _emb_gather(idx3, table)
    return out.reshape(BATCH, N_FIELDS, EMBED_DIM)
